# Initial kernel scaffold; baseline (speedup 1.0000x reference)
import jax
import jax.numpy as jnp
from jax.experimental import pallas as pl


def kernel(pos_seq, pe_k):
    raise NotImplementedError("write your pallas kernel here")


# speedup vs baseline: 1.1424x; 1.1424x over previous
import functools

import jax
import jax.numpy as jnp
from jax import lax
from jax.experimental import pallas as pl
from jax.experimental.pallas import tpu as pltpu
from jax.experimental.pallas import tpu_sc as plsc

_D_MODEL = 2048
_MAXLEN = 4096
_SEQ = 8192

_NW = 32
_BPW = _SEQ // _NW
_CH = 16
_NCH = _BPW // _CH
_L = 16


def _body(pos_hbm, tab_hbm, out_hbm, idx_v, rows0, rows1, sem0, sem1):
    wid = lax.axis_index("s") * 2 + lax.axis_index("c")
    base = wid * _BPW

    pltpu.sync_copy(pos_hbm.at[pl.ds(base, _BPW)], idx_v)

    for i in range(_BPW // _L):
        v = idx_v[pl.ds(i * _L, _L)]
        v = jnp.maximum(jnp.minimum(v, _MAXLEN - 1), -_MAXLEN) + _MAXLEN
        idx_v[pl.ds(i * _L, _L)] = v

    bufs = (rows0, rows1)
    sems = (sem0, sem1)

    def _gather(g, buf, sem):
        pltpu.async_copy(tab_hbm.at[idx_v.at[pl.ds(g * _CH, _CH)]], buf, sem)

    def _drain(g, buf, sem):
        pltpu.make_async_copy(tab_hbm.at[idx_v.at[pl.ds(g * _CH, _CH)]],
                              buf, sem).wait()
        pltpu.sync_copy(buf, out_hbm.at[pl.ds(base + g * _CH, _CH)])

    _gather(0, bufs[0], sems[0])

    @pl.loop(0, _NCH - 1)
    def _(g):
        for b in range(2):
            @pl.when(g % 2 == b)
            def _():
                _gather(g + 1, bufs[1 - b], sems[1 - b])
                _drain(g, bufs[b], sems[b])

    for b in range(2):
        @pl.when((_NCH - 1) % 2 == b)
        def _():
            _drain(_NCH - 1, bufs[b], sems[b])


@jax.jit
def kernel(pos_seq, pe_k):
    run = pl.kernel(
        _body,
        out_type=jax.ShapeDtypeStruct((_SEQ, _D_MODEL), jnp.float32),
        mesh=plsc.VectorSubcoreMesh(core_axis_name="c", subcore_axis_name="s"),
        scratch_types=[
            pltpu.VMEM((_BPW,), jnp.int32),
            pltpu.VMEM((_CH, _D_MODEL), jnp.float32),
            pltpu.VMEM((_CH, _D_MODEL), jnp.float32),
            pltpu.SemaphoreType.DMA,
            pltpu.SemaphoreType.DMA,
        ],
    )
    return run(pos_seq, pe_k)
